# Initial kernel scaffold; baseline (speedup 1.0000x reference)
import jax
import jax.numpy as jnp
from jax.experimental import pallas as pl


def kernel(v_x, v_up_index, v_down_index, v_up_attr, v_down_attr, e_x, e_up_index, e_down_index, e_up_attr, e_down_attr, t_x, t_up_index, t_down_index, t_up_attr, t_down_attr):
    raise NotImplementedError("write your pallas kernel here")


# speedup vs baseline: 2.9614x; 2.9614x over previous
import functools

import jax
import jax.numpy as jnp
from jax import lax
from jax.experimental import pallas as pl
from jax.experimental.pallas import tpu as pltpu
from jax.experimental.pallas import tpu_sc as plsc

_NC = 2
_NS = 16
_NW = _NC * _NS
_C = 80
_NP = 10240


def _sc_partials(xs, srcs, dsts, attrs, zeros):
    N, D = xs[0].shape
    E = attrs[0].shape[0]
    ept = E // _NW
    nchunk = ept // _C
    rpt = _NP // _NS

    mesh = plsc.VectorSubcoreMesh(core_axis_name="c", subcore_axis_name="s",
                                  num_cores=_NC, num_subcores=_NS)

    @functools.partial(
        pl.kernel,
        out_type=jax.ShapeDtypeStruct((3, _NC, _NP, D), jnp.float32),
        mesh=mesh,
        scratch_types=[
            pltpu.VMEM_SHARED((_NP, D), jnp.float32),
            pltpu.VMEM((ept,), jnp.int32),
            pltpu.VMEM((1, _C), jnp.int32),
            pltpu.VMEM((_C, D), jnp.float32),
            pltpu.VMEM((_C, D), jnp.float32),
            pltpu.SemaphoreType.DMA,
        ],
    )
    def body(x0, x1, x2, s0, d0, a0, s1, d1, a1, s2, d2, a2,
             s3, d3, a3, s4, d4, a4, s5, d5, a5, zz, out,
             acc, sidx, didx, xbuf, abuf, sem):
        xs_r = (x0, x1, x2)
        esets = ((s0, d0, a0), (s1, d1, a1), (s2, d2, a2),
                 (s3, d3, a3), (s4, d4, a4), (s5, d5, a5))
        cid = lax.axis_index("c")
        sid = lax.axis_index("s")
        wid = sid * _NC + cid
        r0 = sid * rpt

        for lv in range(3):
            pltpu.sync_copy(zz, acc.at[pl.ds(r0, rpt)])
            plsc.subcore_barrier()

            for src, dst, attr in esets[2 * lv:2 * lv + 2]:
                x = xs_r[lv]
                ebase = pl.multiple_of(wid * ept, 8)
                pltpu.sync_copy(src.at[pl.ds(ebase, ept)], sidx)

                @pl.loop(0, nchunk)
                def chunk(k):
                    off = pl.multiple_of(wid * ept + k * _C, 8)
                    pltpu.sync_copy(dst.at[pl.ds(off, _C)], didx.at[0])
                    pltpu.async_copy(
                        x.at[sidx.at[pl.ds(k * _C, _C)]], xbuf, sem).wait()
                    pltpu.sync_copy(attr.at[pl.ds(off, _C)], abuf)
                    pltpu.sync_copy(xbuf, acc.at[didx.at[0]], add=True)
                    pltpu.sync_copy(abuf, acc.at[didx.at[0]], add=True)

            plsc.subcore_barrier()
            pltpu.sync_copy(acc.at[pl.ds(r0, rpt)],
                            out.at[lv, cid, pl.ds(r0, rpt)])
        return None

    return body(*xs, *[a for es in zip(srcs, dsts, attrs) for a in es], zeros)


def _combine_body(p_ref, o0, o1, o2):
    for v, o in enumerate((o0, o1, o2)):
        o[...] = p_ref[v, 0] + p_ref[v, 1]


def _combine(parts, N):
    D = parts.shape[-1]
    blk = 1000
    return pl.pallas_call(
        _combine_body,
        grid=(N // blk,),
        in_specs=[pl.BlockSpec((3, 2, blk, D), lambda i: (0, 0, i, 0))],
        out_specs=[pl.BlockSpec((blk, D), lambda i: (i, 0))] * 3,
        out_shape=[jax.ShapeDtypeStruct((N, D), jnp.float32)] * 3,
    )(parts)


def kernel(v_x, v_up_index, v_down_index, v_up_attr, v_down_attr,
           e_x, e_up_index, e_down_index, e_up_attr, e_down_attr,
           t_x, t_up_index, t_down_index, t_up_attr, t_down_attr):
    N, D = v_x.shape
    xs = [v_x, e_x, t_x]
    idx = [v_up_index, v_down_index, e_up_index, e_down_index,
           t_up_index, t_down_index]
    srcs = [i[0] for i in idx]
    dsts = [i[1] for i in idx]
    attrs = [v_up_attr, v_down_attr, e_up_attr, e_down_attr,
             t_up_attr, t_down_attr]
    zeros = jnp.zeros((_NP // _NS, D), dtype=jnp.float32)
    parts = _sc_partials(xs, srcs, dsts, attrs, zeros)
    o0, o1, o2 = _combine(parts, N)
    return (o0, o1, o2)


# speedup vs baseline: 5.9417x; 2.0064x over previous
import functools

import jax
import jax.numpy as jnp
from jax import lax
from jax.experimental import pallas as pl
from jax.experimental.pallas import tpu as pltpu
from jax.experimental.pallas import tpu_sc as plsc

_NC = 2
_NS = 16
_NW = _NC * _NS
_C = 40
_NB = 3
_NP = 10240


def _sc_partials(xs, srcs, dsts, attrs, zeros):
    N, D = xs[0].shape
    E = attrs[0].shape[0]
    ept = E // _NW
    nchunk = ept // _C
    rpt = _NP // _NS

    mesh = plsc.VectorSubcoreMesh(core_axis_name="c", subcore_axis_name="s",
                                  num_cores=_NC, num_subcores=_NS)

    @functools.partial(
        pl.kernel,
        out_type=jax.ShapeDtypeStruct((3, _NC, _NP, D), jnp.float32),
        mesh=mesh,
        scratch_types=[
            pltpu.VMEM_SHARED((_NP, D), jnp.float32),
            pltpu.VMEM((ept,), jnp.int32),
            pltpu.VMEM((_NB, _C), jnp.int32),
            pltpu.VMEM((_NB, _C, D), jnp.float32),
            pltpu.VMEM((_NB, _C, D), jnp.float32),
            [pltpu.SemaphoreType.DMA] * _NB,
            [pltpu.SemaphoreType.DMA] * _NB,
        ],
    )
    def body(x0, x1, x2, s0, d0, a0, s1, d1, a1, s2, d2, a2,
             s3, d3, a3, s4, d4, a4, s5, d5, a5, zz, out,
             acc, sidx, didx, xbuf, abuf, fsem, ssem):
        xs_r = (x0, x1, x2)
        esets = ((s0, d0, a0), (s1, d1, a1), (s2, d2, a2),
                 (s3, d3, a3), (s4, d4, a4), (s5, d5, a5))
        cid = lax.axis_index("c")
        sid = lax.axis_index("s")
        wid = sid * _NC + cid
        r0 = sid * rpt
        ngrp = nchunk // _NB

        for lv in range(3):
            pltpu.sync_copy(zz, acc.at[pl.ds(r0, rpt)])
            plsc.subcore_barrier()

            for src, dst, attr in esets[2 * lv:2 * lv + 2]:
                x = xs_r[lv]
                ebase = pl.multiple_of(wid * ept, 8)
                pltpu.sync_copy(src.at[pl.ds(ebase, ept)], sidx)

                def fills(k, b, start):
                    off = pl.multiple_of(wid * ept + k * _C, 8)
                    for cp in (
                        pltpu.make_async_copy(
                            dst.at[pl.ds(off, _C)], didx.at[b], fsem[b]),
                        pltpu.make_async_copy(
                            x.at[sidx.at[pl.ds(k * _C, _C)]], xbuf.at[b],
                            fsem[b]),
                        pltpu.make_async_copy(
                            attr.at[pl.ds(off, _C)], abuf.at[b], fsem[b]),
                    ):
                        cp.start() if start else cp.wait()

                def scats(b, start):
                    if start:
                        pltpu.async_copy(xbuf.at[b], acc.at[didx.at[b]],
                                         ssem[b], add=True)
                        pltpu.async_copy(abuf.at[b], acc.at[didx.at[b]],
                                         ssem[b], add=True)
                    else:
                        for buf in (xbuf, abuf):
                            pltpu.make_async_copy(
                                buf.at[b], acc.at[didx.at[b]], ssem[b]).wait()

                for b in range(_NB):
                    fills(b, b, start=True)

                @pl.loop(0, ngrp - 1)
                def grp(g):
                    for b in range(_NB):
                        fills(g * _NB + b, b, start=False)
                        scats(b, start=True)
                    for b in range(_NB):
                        scats(b, start=False)
                        fills((g + 1) * _NB + b, b, start=True)

                for b in range(_NB):
                    fills((ngrp - 1) * _NB + b, b, start=False)
                    scats(b, start=True)
                for b in range(_NB):
                    scats(b, start=False)
                for k in range(ngrp * _NB, nchunk):
                    fills(k, 0, start=True)
                    fills(k, 0, start=False)
                    scats(0, start=True)
                    scats(0, start=False)

            plsc.subcore_barrier()
            pltpu.sync_copy(acc.at[pl.ds(r0, rpt)],
                            out.at[lv, cid, pl.ds(r0, rpt)])
        return None

    return body(*xs, *[a for es in zip(srcs, dsts, attrs) for a in es], zeros)


def _combine_body(p_ref, o0, o1, o2):
    for v, o in enumerate((o0, o1, o2)):
        o[...] = p_ref[v, 0] + p_ref[v, 1]


def _combine(parts, N):
    D = parts.shape[-1]
    blk = 1000
    return pl.pallas_call(
        _combine_body,
        grid=(N // blk,),
        in_specs=[pl.BlockSpec((3, 2, blk, D), lambda i: (0, 0, i, 0))],
        out_specs=[pl.BlockSpec((blk, D), lambda i: (i, 0))] * 3,
        out_shape=[jax.ShapeDtypeStruct((N, D), jnp.float32)] * 3,
    )(parts)


def kernel(v_x, v_up_index, v_down_index, v_up_attr, v_down_attr,
           e_x, e_up_index, e_down_index, e_up_attr, e_down_attr,
           t_x, t_up_index, t_down_index, t_up_attr, t_down_attr):
    N, D = v_x.shape
    xs = [v_x, e_x, t_x]
    idx = [v_up_index, v_down_index, e_up_index, e_down_index,
           t_up_index, t_down_index]
    srcs = [i[0] for i in idx]
    dsts = [i[1] for i in idx]
    attrs = [v_up_attr, v_down_attr, e_up_attr, e_down_attr,
             t_up_attr, t_down_attr]
    zeros = jnp.zeros((_NP // _NS, D), dtype=jnp.float32)
    parts = _sc_partials(xs, srcs, dsts, attrs, zeros)
    o0, o1, o2 = _combine(parts, N)
    return (o0, o1, o2)


# speedup vs baseline: 6.3477x; 1.0683x over previous
import functools

import jax
import jax.numpy as jnp
from jax import lax
from jax.experimental import pallas as pl
from jax.experimental.pallas import tpu as pltpu
from jax.experimental.pallas import tpu_sc as plsc

_NC = 2
_NS = 16
_NW = _NC * _NS
_C = 80
_NB = 4
_NP = 10240


def _sc_partials(xs, srcs, dsts, attrs, zeros):
    N, D = xs[0].shape
    E = attrs[0].shape[0]
    ept = E // _NW
    nchunk = ept // _C
    rpt = _NP // _NS

    mesh = plsc.VectorSubcoreMesh(core_axis_name="c", subcore_axis_name="s",
                                  num_cores=_NC, num_subcores=_NS)

    @functools.partial(
        pl.kernel,
        out_type=jax.ShapeDtypeStruct((3, _NC, _NP, D), jnp.float32),
        mesh=mesh,
        scratch_types=[
            pltpu.VMEM_SHARED((_NP, D), jnp.float32),
            pltpu.VMEM((_NB, _C), jnp.int32),
            pltpu.VMEM((_NB, _C), jnp.int32),
            pltpu.VMEM((_NB, _C, D), jnp.float32),
            [pltpu.SemaphoreType.DMA] * _NB,
            [pltpu.SemaphoreType.DMA] * _NB,
            [pltpu.SemaphoreType.DMA] * _NB,
        ],
    )
    def body(x0, x1, x2, s0, d0, a0, s1, d1, a1, s2, d2, a2,
             s3, d3, a3, s4, d4, a4, s5, d5, a5, zz, out,
             acc, sidx, didx, abuf, fsem, gsem, ssem):
        xs_r = (x0, x1, x2)
        esets = ((s0, d0, a0), (s1, d1, a1), (s2, d2, a2),
                 (s3, d3, a3), (s4, d4, a4), (s5, d5, a5))
        cid = lax.axis_index("c")
        sid = lax.axis_index("s")
        wid = sid * _NC + cid
        r0 = sid * rpt
        ngrp = nchunk // _NB

        for lv in range(3):
            pltpu.sync_copy(zz, acc.at[pl.ds(r0, rpt)])
            plsc.subcore_barrier()

            for src, dst, attr in esets[2 * lv:2 * lv + 2]:
                x = xs_r[lv]

                def stepA(k, b, first):
                    if not first:
                        pltpu.make_async_copy(
                            abuf.at[b], acc.at[didx.at[b]], ssem[b]).wait()
                    off = pl.multiple_of(wid * ept + k * _C, 8)
                    pltpu.async_copy(src.at[pl.ds(off, _C)], sidx.at[b],
                                     fsem[b])
                    pltpu.async_copy(dst.at[pl.ds(off, _C)], didx.at[b],
                                     fsem[b])
                    pltpu.async_copy(attr.at[pl.ds(off, _C)], abuf.at[b],
                                     fsem[b])

                def stepB(b):
                    for s_r, d_r in ((src, sidx), (dst, didx)):
                        pltpu.make_async_copy(
                            s_r.at[pl.ds(0, _C)], d_r.at[b], fsem[b]).wait()
                    pltpu.make_async_copy(
                        attr.at[pl.ds(0, _C)], abuf.at[b], fsem[b]).wait()
                    pltpu.async_copy(x.at[sidx.at[b]], abuf.at[b], gsem[b],
                                     add=True)

                def stepC(b):
                    pltpu.make_async_copy(
                        x.at[sidx.at[b]], abuf.at[b], gsem[b]).wait()
                    pltpu.async_copy(abuf.at[b], acc.at[didx.at[b]],
                                     ssem[b], add=True)

                n = nchunk
                for t in range(_NB - 1):
                    stepA(t, t % _NB, first=True)
                stepB(0)
                stepC(0)
                stepB(1 % _NB)
                stepA(_NB - 1, (_NB - 1) % _NB, first=True)

                ngrp4 = (n - 4) // _NB
                @pl.loop(0, ngrp4)
                def grp(g):
                    for j in range(_NB):
                        u = 1 + g * _NB + j
                        stepC((1 + j) % _NB)
                        stepB((2 + j) % _NB)
                        stepA(u + 3, j % _NB, first=False)
                for u in range(1 + ngrp4 * _NB, n - 3):
                    stepC(u % _NB)
                    stepB((u + 1) % _NB)
                    stepA(u + 3, (u + 3) % _NB, first=False)
                for u in range(n - 3, n):
                    stepC(u % _NB)
                    if u + 1 < n:
                        stepB((u + 1) % _NB)
                for b in range(_NB):
                    pltpu.make_async_copy(
                        abuf.at[b], acc.at[didx.at[b]], ssem[b]).wait()

            plsc.subcore_barrier()
            pltpu.sync_copy(acc.at[pl.ds(r0, rpt)],
                            out.at[lv, cid, pl.ds(r0, rpt)])
        return None

    return body(*xs, *[a for es in zip(srcs, dsts, attrs) for a in es], zeros)


def _combine_body(p_ref, o0, o1, o2):
    for v, o in enumerate((o0, o1, o2)):
        o[...] = p_ref[v, 0] + p_ref[v, 1]


def _combine(parts, N):
    D = parts.shape[-1]
    blk = 1000
    return pl.pallas_call(
        _combine_body,
        grid=(N // blk,),
        in_specs=[pl.BlockSpec((3, 2, blk, D), lambda i: (0, 0, i, 0))],
        out_specs=[pl.BlockSpec((blk, D), lambda i: (i, 0))] * 3,
        out_shape=[jax.ShapeDtypeStruct((N, D), jnp.float32)] * 3,
    )(parts)


def kernel(v_x, v_up_index, v_down_index, v_up_attr, v_down_attr,
           e_x, e_up_index, e_down_index, e_up_attr, e_down_attr,
           t_x, t_up_index, t_down_index, t_up_attr, t_down_attr):
    N, D = v_x.shape
    xs = [v_x, e_x, t_x]
    idx = [v_up_index, v_down_index, e_up_index, e_down_index,
           t_up_index, t_down_index]
    srcs = [i[0] for i in idx]
    dsts = [i[1] for i in idx]
    attrs = [v_up_attr, v_down_attr, e_up_attr, e_down_attr,
             t_up_attr, t_down_attr]
    zeros = jnp.zeros((_NP // _NS, D), dtype=jnp.float32)
    parts = _sc_partials(xs, srcs, dsts, attrs, zeros)
    o0, o1, o2 = _combine(parts, N)
    return (o0, o1, o2)
